# baseline (speedup 1.0000x reference)
import jax
import jax.numpy as jnp
from jax.experimental import pallas as pl
from jax.experimental.pallas import tpu as pltpu

_ROWS = 16384
_FEAT = 768
_BM = 2048


def _concat_kernel(v_ref, t_ref, o_ref):
    o_ref[:, 0:_FEAT] = v_ref[...]
    o_ref[:, _FEAT : 2 * _FEAT] = t_ref[...]


def kernel(vision_input, text_input):
    out_shape = jax.ShapeDtypeStruct((_ROWS, 2 * _FEAT), vision_input.dtype)
    return pl.pallas_call(
        _concat_kernel,
        grid=(_ROWS // _BM,),
        in_specs=[
            pl.BlockSpec((_BM, _FEAT), lambda i: (i, 0)),
            pl.BlockSpec((_BM, _FEAT), lambda i: (i, 0)),
        ],
        out_specs=pl.BlockSpec((_BM, 2 * _FEAT), lambda i: (i, 0)),
        out_shape=out_shape,
        compiler_params=pltpu.CompilerParams(dimension_semantics=("parallel",)),
    )(vision_input, text_input)


# speedup vs baseline: 1.0004x; 1.0004x over previous
import jax
import jax.numpy as jnp
from jax.experimental import pallas as pl
from jax.experimental.pallas import tpu as pltpu

_ROWS = 16384
_FEAT = 768
_BM = 2048


def _concat_kernel(v_ref, t_ref, o_ref):
    o_ref[:, 0:_FEAT] = v_ref[...]
    o_ref[:, _FEAT : 2 * _FEAT] = t_ref[...]


def kernel(vision_input, text_input):
    out_shape = jax.ShapeDtypeStruct((_ROWS, 2 * _FEAT), vision_input.dtype)
    return pl.pallas_call(
        _concat_kernel,
        grid=(_ROWS // _BM,),
        in_specs=[
            pl.BlockSpec((_BM, _FEAT), lambda i: (i, 0)),
            pl.BlockSpec((_BM, _FEAT), lambda i: (i, 0)),
        ],
        out_specs=pl.BlockSpec((_BM, 2 * _FEAT), lambda i: (i, 0)),
        out_shape=out_shape,
    )(vision_input, text_input)
